# Initial kernel scaffold; baseline (speedup 1.0000x reference)
import jax
import jax.numpy as jnp
from jax.experimental import pallas as pl


def kernel(vectorized_text, table, W1, b1, W2, b2):
    raise NotImplementedError("write your pallas kernel here")


# speedup vs baseline: 2.7104x; 2.7104x over previous
import functools

import jax
import jax.numpy as jnp
from jax import lax
from jax.experimental import pallas as pl
from jax.experimental.pallas import tpu as pltpu
from jax.experimental.pallas import tpu_sc as plsc

BATCH = 16384
HIST = 50
DIM = 64

NUM_CORES = 2
NUM_SUBCORES = 16
NUM_WORKERS = NUM_CORES * NUM_SUBCORES
BAGS_PER_WORKER = BATCH // NUM_WORKERS
IDX_PER_WORKER = BAGS_PER_WORKER * HIST

CHUNK_BAGS = 8
CHUNK_ROWS = CHUNK_BAGS * HIST
SUB_ROWS = 80
SUBS_PER_CHUNK = CHUNK_ROWS // SUB_ROWS
NUM_CHUNKS = BAGS_PER_WORKER // CHUNK_BAGS


def _sc_embedding_sum(idx_flat, table):
    mesh = plsc.VectorSubcoreMesh(core_axis_name="c", subcore_axis_name="s")

    @functools.partial(
        pl.kernel,
        mesh=mesh,
        compiler_params=pltpu.CompilerParams(use_tc_tiling_on_sc=False),
        out_type=jax.ShapeDtypeStruct((BATCH, DIM), jnp.float32),
        scratch_types=[
            pltpu.VMEM((IDX_PER_WORKER,), jnp.int32),
            pltpu.VMEM((CHUNK_ROWS, DIM), jnp.float32),
            pltpu.VMEM((CHUNK_ROWS, DIM), jnp.float32),
            pltpu.VMEM((CHUNK_BAGS, DIM), jnp.float32),
            pltpu.SemaphoreType.DMA,
            pltpu.SemaphoreType.DMA,
        ],
    )
    def k(idx_hbm, table_hbm, out_hbm, idx_v, rows0, rows1, outb, sem0, sem1):
        wid = lax.axis_index("s") * NUM_CORES + lax.axis_index("c")
        bag0 = wid * BAGS_PER_WORKER

        pltpu.sync_copy(idx_hbm.at[pl.ds(bag0 * HIST, IDX_PER_WORKER)], idx_v)

        def start_gather(c, buf, sem):
            for k_ in range(SUBS_PER_CHUNK):
                pltpu.make_async_copy(
                    table_hbm.at[idx_v.at[pl.ds(c * CHUNK_ROWS + k_ * SUB_ROWS,
                                                SUB_ROWS)]],
                    buf.at[pl.ds(k_ * SUB_ROWS, SUB_ROWS)],
                    sem,
                ).start()

        def wait_gather(c, buf, sem):
            for k_ in range(SUBS_PER_CHUNK):
                pltpu.make_async_copy(
                    table_hbm.at[idx_v.at[pl.ds(c * CHUNK_ROWS + k_ * SUB_ROWS,
                                                SUB_ROWS)]],
                    buf.at[pl.ds(k_ * SUB_ROWS, SUB_ROWS)],
                    sem,
                ).wait()

        def accumulate(c, buf):
            for g in range(CHUNK_BAGS):
                def body(r, acc):
                    row = g * HIST + r
                    return tuple(
                        acc[j] + buf[row, pl.ds(j * 16, 16)] for j in range(4)
                    )
                zero = jnp.zeros((16,), jnp.float32)
                acc = lax.fori_loop(0, HIST, body, (zero, zero, zero, zero))
                for j in range(4):
                    outb[g, pl.ds(j * 16, 16)] = acc[j]
            pltpu.sync_copy(outb,
                            out_hbm.at[pl.ds(bag0 + c * CHUNK_BAGS, CHUNK_BAGS)])

        start_gather(0, rows0, sem0)
        start_gather(1, rows1, sem1)

        def loop_body(c2, carry):
            for b, (buf, sem) in enumerate(((rows0, sem0), (rows1, sem1))):
                c = c2 * 2 + b
                wait_gather(c, buf, sem)
                accumulate(c, buf)
                nxt = c + 2

                @pl.when(nxt < NUM_CHUNKS)
                def _():
                    start_gather(nxt, buf, sem)
            return carry

        lax.fori_loop(0, NUM_CHUNKS // 2, loop_body, 0)

    return k(idx_flat, table)


def _tc_mlp_kernel(summed_ref, idx_ref, w1_ref, b1_ref, w2_ref, b2_ref, out_ref):
    counts = jnp.sum((idx_ref[...] != 0).astype(jnp.float32), axis=1,
                     keepdims=True)
    pooled = summed_ref[...] / jnp.maximum(counts, 1.0)
    h = jnp.maximum(
        jnp.dot(pooled, w1_ref[...], preferred_element_type=jnp.float32)
        + b1_ref[...], 0.0)
    out_ref[...] = (
        jnp.dot(h, w2_ref[...], preferred_element_type=jnp.float32)
        + b2_ref[...])


def _tc_mlp(summed, idx, W1, b1, W2, b2):
    blk = 2048
    grid = (BATCH // blk,)
    return pl.pallas_call(
        _tc_mlp_kernel,
        grid=grid,
        in_specs=[
            pl.BlockSpec((blk, DIM), lambda i: (i, 0)),
            pl.BlockSpec((blk, HIST), lambda i: (i, 0)),
            pl.BlockSpec(W1.shape, lambda i: (0, 0)),
            pl.BlockSpec((1, b1.shape[1]), lambda i: (0, 0)),
            pl.BlockSpec(W2.shape, lambda i: (0, 0)),
            pl.BlockSpec((1, b2.shape[1]), lambda i: (0, 0)),
        ],
        out_specs=pl.BlockSpec((blk, DIM), lambda i: (i, 0)),
        out_shape=jax.ShapeDtypeStruct((BATCH, DIM), jnp.float32),
    )(summed, idx, W1, b1, W2, b2)


def kernel(vectorized_text, table, W1, b1, W2, b2):
    idx = vectorized_text.astype(jnp.int32)
    summed = _sc_embedding_sum(idx.reshape(-1), table)
    return _tc_mlp(summed, idx, W1, b1.reshape(1, -1), W2, b2.reshape(1, -1))


# speedup vs baseline: 2.7243x; 1.0052x over previous
import functools

import jax
import jax.numpy as jnp
from jax import lax
from jax.experimental import pallas as pl
from jax.experimental.pallas import tpu as pltpu
from jax.experimental.pallas import tpu_sc as plsc

BATCH = 16384
HIST = 50
DIM = 64
VOCAB = 1000000

NUM_CORES = 2
NUM_SUBCORES = 16
NUM_WORKERS = NUM_CORES * NUM_SUBCORES
BAGS_PER_WORKER = BATCH // NUM_WORKERS
IDX_PER_WORKER = BAGS_PER_WORKER * HIST

CHUNK_BAGS = 8
CHUNK_ROWS = CHUNK_BAGS * HIST
SUB_ROWS = 80
SUBS_PER_CHUNK = CHUNK_ROWS // SUB_ROWS
NUM_CHUNKS = BAGS_PER_WORKER // CHUNK_BAGS

CONV_BLK = 2048


def _convert_kernel(tt_ref, out_ref):
    ident = (lax.broadcasted_iota(jnp.int32, (DIM, DIM), 0)
             == lax.broadcasted_iota(jnp.int32, (DIM, DIM), 1)
             ).astype(jnp.float32)
    xt = lax.dot_general(tt_ref[...], ident, (((0,), (0,)), ((), ())),
                         preferred_element_type=jnp.float32)
    out_ref[...] = jnp.concatenate([xt, xt], axis=1)


def _tc_convert(table):
    tt = table.T
    grid = (pl.cdiv(VOCAB, CONV_BLK),)
    return pl.pallas_call(
        _convert_kernel,
        grid=grid,
        in_specs=[pl.BlockSpec((DIM, CONV_BLK), lambda i: (0, i))],
        out_specs=pl.BlockSpec((CONV_BLK, 2 * DIM), lambda i: (i, 0)),
        out_shape=jax.ShapeDtypeStruct((VOCAB, 2 * DIM), jnp.float32),
    )(tt)


def _sc_embedding_sum(idx_flat, table2):
    mesh = plsc.VectorSubcoreMesh(core_axis_name="c", subcore_axis_name="s")

    @functools.partial(
        pl.kernel,
        mesh=mesh,
        out_type=jax.ShapeDtypeStruct((BATCH, 2 * DIM), jnp.float32),
        scratch_types=[
            pltpu.VMEM((IDX_PER_WORKER,), jnp.int32),
            pltpu.VMEM((CHUNK_ROWS, 2 * DIM), jnp.float32),
            pltpu.VMEM((CHUNK_ROWS, 2 * DIM), jnp.float32),
            pltpu.VMEM((CHUNK_BAGS, 2 * DIM), jnp.float32),
            pltpu.SemaphoreType.DMA,
            pltpu.SemaphoreType.DMA,
        ],
    )
    def k(idx_hbm, table_hbm, out_hbm, idx_v, rows0, rows1, outb, sem0, sem1):
        wid = lax.axis_index("s") * NUM_CORES + lax.axis_index("c")
        bag0 = wid * BAGS_PER_WORKER

        pltpu.sync_copy(idx_hbm.at[pl.ds(bag0 * HIST, IDX_PER_WORKER)], idx_v)

        def start_gather(c, buf, sem):
            for k_ in range(SUBS_PER_CHUNK):
                pltpu.make_async_copy(
                    table_hbm.at[idx_v.at[pl.ds(c * CHUNK_ROWS + k_ * SUB_ROWS,
                                                SUB_ROWS)]],
                    buf.at[pl.ds(k_ * SUB_ROWS, SUB_ROWS)],
                    sem,
                ).start()

        def wait_gather(c, buf, sem):
            for k_ in range(SUBS_PER_CHUNK):
                pltpu.make_async_copy(
                    table_hbm.at[idx_v.at[pl.ds(c * CHUNK_ROWS + k_ * SUB_ROWS,
                                                SUB_ROWS)]],
                    buf.at[pl.ds(k_ * SUB_ROWS, SUB_ROWS)],
                    sem,
                ).wait()

        def accumulate(c, buf):
            for g in range(CHUNK_BAGS):
                def body(r, acc):
                    row = g * HIST + r
                    return tuple(
                        acc[j] + buf[row, pl.ds(j * 16, 16)] for j in range(4)
                    )
                zero = jnp.zeros((16,), jnp.float32)
                acc = lax.fori_loop(0, HIST, body, (zero, zero, zero, zero))
                for j in range(4):
                    outb[g, pl.ds(j * 16, 16)] = acc[j]
            pltpu.sync_copy(outb,
                            out_hbm.at[pl.ds(bag0 + c * CHUNK_BAGS, CHUNK_BAGS)])

        start_gather(0, rows0, sem0)
        start_gather(1, rows1, sem1)

        def loop_body(c2, carry):
            for b, (buf, sem) in enumerate(((rows0, sem0), (rows1, sem1))):
                c = c2 * 2 + b
                wait_gather(c, buf, sem)
                accumulate(c, buf)
                nxt = c + 2

                @pl.when(nxt < NUM_CHUNKS)
                def _():
                    start_gather(nxt, buf, sem)
            return carry

        lax.fori_loop(0, NUM_CHUNKS // 2, loop_body, 0)

    return k(idx_flat, table2)


def _tc_mlp_kernel(summed_ref, idx_ref, w1_ref, b1_ref, w2_ref, b2_ref, out_ref):
    counts = jnp.sum((idx_ref[...] != 0).astype(jnp.float32), axis=1,
                     keepdims=True)
    pooled = summed_ref[...][:, :DIM] / jnp.maximum(counts, 1.0)
    h = jnp.maximum(
        jnp.dot(pooled, w1_ref[...], preferred_element_type=jnp.float32)
        + b1_ref[...], 0.0)
    out_ref[...] = (
        jnp.dot(h, w2_ref[...], preferred_element_type=jnp.float32)
        + b2_ref[...])


def _tc_mlp(summed, idx, W1, b1, W2, b2):
    blk = 2048
    grid = (BATCH // blk,)
    return pl.pallas_call(
        _tc_mlp_kernel,
        grid=grid,
        in_specs=[
            pl.BlockSpec((blk, 2 * DIM), lambda i: (i, 0)),
            pl.BlockSpec((blk, HIST), lambda i: (i, 0)),
            pl.BlockSpec(W1.shape, lambda i: (0, 0)),
            pl.BlockSpec((1, b1.shape[1]), lambda i: (0, 0)),
            pl.BlockSpec(W2.shape, lambda i: (0, 0)),
            pl.BlockSpec((1, b2.shape[1]), lambda i: (0, 0)),
        ],
        out_specs=pl.BlockSpec((blk, DIM), lambda i: (i, 0)),
        out_shape=jax.ShapeDtypeStruct((BATCH, DIM), jnp.float32),
    )(summed, idx, W1, b1, W2, b2)


def kernel(vectorized_text, table, W1, b1, W2, b2):
    idx = vectorized_text.astype(jnp.int32)
    table2 = _tc_convert(table)
    summed = _sc_embedding_sum(idx.reshape(-1), table2)
    return _tc_mlp(summed, idx, W1, b1.reshape(1, -1), W2, b2.reshape(1, -1))


# speedup vs baseline: 3.7090x; 1.3614x over previous
import functools

import jax
import jax.numpy as jnp
from jax import lax
from jax.experimental import pallas as pl
from jax.experimental.pallas import tpu as pltpu
from jax.experimental.pallas import tpu_sc as plsc

BATCH = 16384
HIST = 50
DIM = 64
VOCAB = 1000000

NUM_CORES = 2
NUM_SUBCORES = 16
NUM_WORKERS = NUM_CORES * NUM_SUBCORES
BAGS_PER_WORKER = BATCH // NUM_WORKERS
IDX_PER_WORKER = BAGS_PER_WORKER * HIST

CHUNK_BAGS = 8
CHUNK_ROWS = CHUNK_BAGS * HIST
SUB_ROWS = 80
SUBS_PER_CHUNK = CHUNK_ROWS // SUB_ROWS
NUM_CHUNKS = BAGS_PER_WORKER // CHUNK_BAGS

CONV_BLK = 8192


def _convert_kernel(tt_ref, out_ref):
    d = lax.broadcasted_iota(jnp.int32, (DIM, DIM), 0)
    o = lax.broadcasted_iota(jnp.int32, (DIM, DIM), 1)
    ident = (d == o).astype(jnp.float32)
    xt = lax.dot_general(tt_ref[...], ident, (((0,), (0,)), ((), ())),
                         preferred_element_type=jnp.float32)
    out_ref[...] = jnp.concatenate([xt, xt], axis=1)


def _tc_convert(table):
    tt = table.T
    grid = (pl.cdiv(VOCAB, CONV_BLK),)
    return pl.pallas_call(
        _convert_kernel,
        grid=grid,
        in_specs=[pl.BlockSpec((DIM, CONV_BLK), lambda i: (0, i))],
        out_specs=pl.BlockSpec((CONV_BLK, 2 * DIM), lambda i: (i, 0)),
        out_shape=jax.ShapeDtypeStruct((VOCAB, 2 * DIM), jnp.float32),
    )(tt)


def _sc_embedding_sum(idx_flat, table2):
    mesh = plsc.VectorSubcoreMesh(core_axis_name="c", subcore_axis_name="s")

    @functools.partial(
        pl.kernel,
        mesh=mesh,
        out_type=jax.ShapeDtypeStruct((BATCH, 2 * DIM), jnp.float32),
        scratch_types=[
            pltpu.VMEM((IDX_PER_WORKER,), jnp.int32),
            pltpu.VMEM((CHUNK_ROWS, 2 * DIM), jnp.float32),
            pltpu.VMEM((CHUNK_ROWS, 2 * DIM), jnp.float32),
            pltpu.VMEM((CHUNK_BAGS, 2 * DIM), jnp.float32),
            pltpu.SemaphoreType.DMA,
            pltpu.SemaphoreType.DMA,
        ],
    )
    def k(idx_hbm, table_hbm, out_hbm, idx_v, rows0, rows1, outb, sem0, sem1):
        wid = lax.axis_index("s") * NUM_CORES + lax.axis_index("c")
        bag0 = wid * BAGS_PER_WORKER

        pltpu.sync_copy(idx_hbm.at[pl.ds(bag0 * HIST, IDX_PER_WORKER)], idx_v)

        def start_gather(c, buf, sem):
            for k_ in range(SUBS_PER_CHUNK):
                pltpu.make_async_copy(
                    table_hbm.at[idx_v.at[pl.ds(c * CHUNK_ROWS + k_ * SUB_ROWS,
                                                SUB_ROWS)]],
                    buf.at[pl.ds(k_ * SUB_ROWS, SUB_ROWS)],
                    sem,
                ).start()

        def wait_gather(c, buf, sem):
            for k_ in range(SUBS_PER_CHUNK):
                pltpu.make_async_copy(
                    table_hbm.at[idx_v.at[pl.ds(c * CHUNK_ROWS + k_ * SUB_ROWS,
                                                SUB_ROWS)]],
                    buf.at[pl.ds(k_ * SUB_ROWS, SUB_ROWS)],
                    sem,
                ).wait()

        def accumulate(c, buf):
            for g in range(CHUNK_BAGS):
                def body(r, acc):
                    row = g * HIST + r
                    return tuple(
                        acc[j] + buf[row, pl.ds(j * 16, 16)] for j in range(4)
                    )
                zero = jnp.zeros((16,), jnp.float32)
                acc = lax.fori_loop(0, HIST, body, (zero, zero, zero, zero))
                for j in range(4):
                    outb[g, pl.ds(j * 16, 16)] = acc[j]
            pltpu.sync_copy(outb,
                            out_hbm.at[pl.ds(bag0 + c * CHUNK_BAGS, CHUNK_BAGS)])

        start_gather(0, rows0, sem0)
        start_gather(1, rows1, sem1)

        def loop_body(c2, carry):
            for b, (buf, sem) in enumerate(((rows0, sem0), (rows1, sem1))):
                c = c2 * 2 + b
                wait_gather(c, buf, sem)
                accumulate(c, buf)
                nxt = c + 2

                @pl.when(nxt < NUM_CHUNKS)
                def _():
                    start_gather(nxt, buf, sem)
            return carry

        lax.fori_loop(0, NUM_CHUNKS // 2, loop_body, 0)

    return k(idx_flat, table2)


def _tc_mlp_kernel(summed_ref, idx_ref, w1_ref, b1_ref, w2_ref, b2_ref, out_ref):
    counts = jnp.sum((idx_ref[...] != 0).astype(jnp.float32), axis=1,
                     keepdims=True)
    pooled = summed_ref[...][:, :DIM] / jnp.maximum(counts, 1.0)
    h = jnp.maximum(
        jnp.dot(pooled, w1_ref[...], preferred_element_type=jnp.float32)
        + b1_ref[...], 0.0)
    out_ref[...] = (
        jnp.dot(h, w2_ref[...], preferred_element_type=jnp.float32)
        + b2_ref[...])


def _tc_mlp(summed, idx, W1, b1, W2, b2):
    blk = 2048
    grid = (BATCH // blk,)
    return pl.pallas_call(
        _tc_mlp_kernel,
        grid=grid,
        in_specs=[
            pl.BlockSpec((blk, 2 * DIM), lambda i: (i, 0)),
            pl.BlockSpec((blk, HIST), lambda i: (i, 0)),
            pl.BlockSpec(W1.shape, lambda i: (0, 0)),
            pl.BlockSpec((1, b1.shape[1]), lambda i: (0, 0)),
            pl.BlockSpec(W2.shape, lambda i: (0, 0)),
            pl.BlockSpec((1, b2.shape[1]), lambda i: (0, 0)),
        ],
        out_specs=pl.BlockSpec((blk, DIM), lambda i: (i, 0)),
        out_shape=jax.ShapeDtypeStruct((BATCH, DIM), jnp.float32),
    )(summed, idx, W1, b1, W2, b2)


def kernel(vectorized_text, table, W1, b1, W2, b2):
    idx = vectorized_text.astype(jnp.int32)
    table2 = _tc_convert(table)
    summed = _sc_embedding_sum(idx.reshape(-1), table2)
    return _tc_mlp(summed, idx, W1, b1.reshape(1, -1), W2, b2.reshape(1, -1))


# speedup vs baseline: 3.9505x; 1.0651x over previous
import functools

import jax
import jax.numpy as jnp
from jax import lax
from jax.experimental import pallas as pl
from jax.experimental.pallas import tpu as pltpu
from jax.experimental.pallas import tpu_sc as plsc

BATCH = 16384
HIST = 50
DIM = 64
VOCAB = 1000000

NUM_CORES = 2
NUM_SUBCORES = 16
NUM_WORKERS = NUM_CORES * NUM_SUBCORES
BAGS_PER_WORKER = BATCH // NUM_WORKERS
IDX_PER_WORKER = BAGS_PER_WORKER * HIST

CHUNK_BAGS = 8
CHUNK_ROWS = CHUNK_BAGS * HIST
SUB_ROWS = 80
SUBS_PER_CHUNK = CHUNK_ROWS // SUB_ROWS
NUM_CHUNKS = BAGS_PER_WORKER // CHUNK_BAGS

CONV_BLK = 16384


def _convert_kernel(tt_ref, out_ref):
    d = lax.broadcasted_iota(jnp.int32, (DIM, DIM), 0)
    o = lax.broadcasted_iota(jnp.int32, (DIM, DIM), 1)
    ident = (d == o).astype(jnp.float32)
    xt = lax.dot_general(tt_ref[...], ident, (((0,), (0,)), ((), ())),
                         preferred_element_type=jnp.float32)
    out_ref[...] = jnp.concatenate([xt, xt], axis=1)


def _tc_convert(table):
    tt = table.T
    grid = (pl.cdiv(VOCAB, CONV_BLK),)
    return pl.pallas_call(
        _convert_kernel,
        grid=grid,
        in_specs=[pl.BlockSpec((DIM, CONV_BLK), lambda i: (0, i))],
        out_specs=pl.BlockSpec((CONV_BLK, 2 * DIM), lambda i: (i, 0)),
        out_shape=jax.ShapeDtypeStruct((VOCAB, 2 * DIM), jnp.float32),
    )(tt)


def _sc_embedding_sum(idx_flat, table2):
    mesh = plsc.VectorSubcoreMesh(core_axis_name="c", subcore_axis_name="s")

    @functools.partial(
        pl.kernel,
        mesh=mesh,
        out_type=jax.ShapeDtypeStruct((BATCH, 2 * DIM), jnp.float32),
        scratch_types=[
            pltpu.VMEM((IDX_PER_WORKER,), jnp.int32),
            pltpu.VMEM((CHUNK_ROWS, 2 * DIM), jnp.float32),
            pltpu.VMEM((CHUNK_ROWS, 2 * DIM), jnp.float32),
            pltpu.VMEM((CHUNK_BAGS, 2 * DIM), jnp.float32),
            pltpu.SemaphoreType.DMA,
            pltpu.SemaphoreType.DMA,
        ],
    )
    def k(idx_hbm, table_hbm, out_hbm, idx_v, rows0, rows1, outb, sem0, sem1):
        wid = lax.axis_index("s") * NUM_CORES + lax.axis_index("c")
        bag0 = wid * BAGS_PER_WORKER

        pltpu.sync_copy(idx_hbm.at[pl.ds(bag0 * HIST, IDX_PER_WORKER)], idx_v)

        def start_gather(c, buf, sem):
            for k_ in range(SUBS_PER_CHUNK):
                pltpu.make_async_copy(
                    table_hbm.at[idx_v.at[pl.ds(c * CHUNK_ROWS + k_ * SUB_ROWS,
                                                SUB_ROWS)]],
                    buf.at[pl.ds(k_ * SUB_ROWS, SUB_ROWS)],
                    sem,
                ).start()

        def wait_gather(c, buf, sem):
            for k_ in range(SUBS_PER_CHUNK):
                pltpu.make_async_copy(
                    table_hbm.at[idx_v.at[pl.ds(c * CHUNK_ROWS + k_ * SUB_ROWS,
                                                SUB_ROWS)]],
                    buf.at[pl.ds(k_ * SUB_ROWS, SUB_ROWS)],
                    sem,
                ).wait()

        def accumulate(c, buf):
            for g in range(CHUNK_BAGS):
                def body(r, acc):
                    row = g * HIST + r
                    return tuple(
                        acc[j] + buf[row, pl.ds(j * 16, 16)] for j in range(4)
                    )
                zero = jnp.zeros((16,), jnp.float32)
                acc = lax.fori_loop(0, HIST, body, (zero, zero, zero, zero))
                for j in range(4):
                    outb[g, pl.ds(j * 16, 16)] = acc[j]
            pltpu.sync_copy(outb,
                            out_hbm.at[pl.ds(bag0 + c * CHUNK_BAGS, CHUNK_BAGS)])

        start_gather(0, rows0, sem0)
        start_gather(1, rows1, sem1)

        def loop_body(c2, carry):
            for b, (buf, sem) in enumerate(((rows0, sem0), (rows1, sem1))):
                c = c2 * 2 + b
                wait_gather(c, buf, sem)
                accumulate(c, buf)
                nxt = c + 2

                @pl.when(nxt < NUM_CHUNKS)
                def _():
                    start_gather(nxt, buf, sem)
            return carry

        lax.fori_loop(0, NUM_CHUNKS // 2, loop_body, 0)

    return k(idx_flat, table2)


def _tc_mlp_kernel(summed_ref, idx_ref, w1_ref, b1_ref, w2_ref, b2_ref, out_ref):
    counts = jnp.sum((idx_ref[...] != 0).astype(jnp.float32), axis=1,
                     keepdims=True)
    pooled = summed_ref[...][:, :DIM] / jnp.maximum(counts, 1.0)
    h = jnp.maximum(
        jnp.dot(pooled, w1_ref[...], preferred_element_type=jnp.float32)
        + b1_ref[...], 0.0)
    out_ref[...] = (
        jnp.dot(h, w2_ref[...], preferred_element_type=jnp.float32)
        + b2_ref[...])


def _tc_mlp(summed, idx, W1, b1, W2, b2):
    blk = 2048
    grid = (BATCH // blk,)
    return pl.pallas_call(
        _tc_mlp_kernel,
        grid=grid,
        in_specs=[
            pl.BlockSpec((blk, 2 * DIM), lambda i: (i, 0)),
            pl.BlockSpec((blk, HIST), lambda i: (i, 0)),
            pl.BlockSpec(W1.shape, lambda i: (0, 0)),
            pl.BlockSpec((1, b1.shape[1]), lambda i: (0, 0)),
            pl.BlockSpec(W2.shape, lambda i: (0, 0)),
            pl.BlockSpec((1, b2.shape[1]), lambda i: (0, 0)),
        ],
        out_specs=pl.BlockSpec((blk, DIM), lambda i: (i, 0)),
        out_shape=jax.ShapeDtypeStruct((BATCH, DIM), jnp.float32),
    )(summed, idx, W1, b1, W2, b2)


def kernel(vectorized_text, table, W1, b1, W2, b2):
    idx = vectorized_text.astype(jnp.int32)
    table2 = _tc_convert(table)
    summed = _sc_embedding_sum(idx.reshape(-1), table2)
    return _tc_mlp(summed, idx, W1, b1.reshape(1, -1), W2, b2.reshape(1, -1))


# speedup vs baseline: 4.0316x; 1.0205x over previous
import functools

import jax
import jax.numpy as jnp
from jax import lax
from jax.experimental import pallas as pl
from jax.experimental.pallas import tpu as pltpu
from jax.experimental.pallas import tpu_sc as plsc

BATCH = 16384
HIST = 50
DIM = 64
VOCAB = 1000000

NUM_CORES = 2
NUM_SUBCORES = 16
NUM_WORKERS = NUM_CORES * NUM_SUBCORES
BAGS_PER_WORKER = BATCH // NUM_WORKERS
IDX_PER_WORKER = BAGS_PER_WORKER * HIST

CHUNK_BAGS = 8
CHUNK_ROWS = CHUNK_BAGS * HIST
SUB_ROWS = 80
SUBS_PER_CHUNK = CHUNK_ROWS // SUB_ROWS
NUM_CHUNKS = BAGS_PER_WORKER // CHUNK_BAGS

CONV_BLK = 24576


def _convert_kernel(tt_ref, out_ref):
    d = lax.broadcasted_iota(jnp.int32, (DIM, DIM), 0)
    o = lax.broadcasted_iota(jnp.int32, (DIM, DIM), 1)
    ident = (d == o).astype(jnp.float32)
    xt = lax.dot_general(tt_ref[...], ident, (((0,), (0,)), ((), ())),
                         preferred_element_type=jnp.float32)
    out_ref[...] = jnp.concatenate([xt, xt], axis=1)


def _tc_convert(table):
    tt = table.T
    grid = (pl.cdiv(VOCAB, CONV_BLK),)
    return pl.pallas_call(
        _convert_kernel,
        grid=grid,
        in_specs=[pl.BlockSpec((DIM, CONV_BLK), lambda i: (0, i))],
        out_specs=pl.BlockSpec((CONV_BLK, 2 * DIM), lambda i: (i, 0)),
        out_shape=jax.ShapeDtypeStruct((VOCAB, 2 * DIM), jnp.float32),
    )(tt)


def _sc_embedding_sum(idx_flat, table2):
    mesh = plsc.VectorSubcoreMesh(core_axis_name="c", subcore_axis_name="s")

    @functools.partial(
        pl.kernel,
        mesh=mesh,
        out_type=jax.ShapeDtypeStruct((BATCH, 2 * DIM), jnp.float32),
        scratch_types=[
            pltpu.VMEM((IDX_PER_WORKER,), jnp.int32),
            pltpu.VMEM((CHUNK_ROWS, 2 * DIM), jnp.float32),
            pltpu.VMEM((CHUNK_ROWS, 2 * DIM), jnp.float32),
            pltpu.VMEM((CHUNK_BAGS, 2 * DIM), jnp.float32),
            pltpu.SemaphoreType.DMA,
            pltpu.SemaphoreType.DMA,
        ],
    )
    def k(idx_hbm, table_hbm, out_hbm, idx_v, rows0, rows1, outb, sem0, sem1):
        wid = lax.axis_index("s") * NUM_CORES + lax.axis_index("c")
        bag0 = wid * BAGS_PER_WORKER

        pltpu.sync_copy(idx_hbm.at[pl.ds(bag0 * HIST, IDX_PER_WORKER)], idx_v)

        def start_gather(c, buf, sem):
            for k_ in range(SUBS_PER_CHUNK):
                pltpu.make_async_copy(
                    table_hbm.at[idx_v.at[pl.ds(c * CHUNK_ROWS + k_ * SUB_ROWS,
                                                SUB_ROWS)]],
                    buf.at[pl.ds(k_ * SUB_ROWS, SUB_ROWS)],
                    sem,
                ).start()

        def wait_gather(c, buf, sem):
            for k_ in range(SUBS_PER_CHUNK):
                pltpu.make_async_copy(
                    table_hbm.at[idx_v.at[pl.ds(c * CHUNK_ROWS + k_ * SUB_ROWS,
                                                SUB_ROWS)]],
                    buf.at[pl.ds(k_ * SUB_ROWS, SUB_ROWS)],
                    sem,
                ).wait()

        def accumulate(c, buf):
            for g in range(CHUNK_BAGS):
                def body(r, acc):
                    row = g * HIST + r
                    return tuple(
                        acc[j] + buf[row, pl.ds(j * 16, 16)] for j in range(4)
                    )
                zero = jnp.zeros((16,), jnp.float32)
                acc = lax.fori_loop(0, HIST, body, (zero, zero, zero, zero))
                for j in range(4):
                    outb[g, pl.ds(j * 16, 16)] = acc[j]
            pltpu.sync_copy(outb,
                            out_hbm.at[pl.ds(bag0 + c * CHUNK_BAGS, CHUNK_BAGS)])

        start_gather(0, rows0, sem0)
        start_gather(1, rows1, sem1)

        def loop_body(c2, carry):
            for b, (buf, sem) in enumerate(((rows0, sem0), (rows1, sem1))):
                c = c2 * 2 + b
                wait_gather(c, buf, sem)
                accumulate(c, buf)
                nxt = c + 2

                @pl.when(nxt < NUM_CHUNKS)
                def _():
                    start_gather(nxt, buf, sem)
            return carry

        lax.fori_loop(0, NUM_CHUNKS // 2, loop_body, 0)

    return k(idx_flat, table2)


def _tc_mlp_kernel(summed_ref, idx_ref, w1_ref, b1_ref, w2_ref, b2_ref, out_ref):
    counts = jnp.sum((idx_ref[...] != 0).astype(jnp.float32), axis=1,
                     keepdims=True)
    pooled = summed_ref[...][:, :DIM] / jnp.maximum(counts, 1.0)
    h = jnp.maximum(
        jnp.dot(pooled, w1_ref[...], preferred_element_type=jnp.float32)
        + b1_ref[...], 0.0)
    out_ref[...] = (
        jnp.dot(h, w2_ref[...], preferred_element_type=jnp.float32)
        + b2_ref[...])


def _tc_mlp(summed, idx, W1, b1, W2, b2):
    blk = 2048
    grid = (BATCH // blk,)
    return pl.pallas_call(
        _tc_mlp_kernel,
        grid=grid,
        in_specs=[
            pl.BlockSpec((blk, 2 * DIM), lambda i: (i, 0)),
            pl.BlockSpec((blk, HIST), lambda i: (i, 0)),
            pl.BlockSpec(W1.shape, lambda i: (0, 0)),
            pl.BlockSpec((1, b1.shape[1]), lambda i: (0, 0)),
            pl.BlockSpec(W2.shape, lambda i: (0, 0)),
            pl.BlockSpec((1, b2.shape[1]), lambda i: (0, 0)),
        ],
        out_specs=pl.BlockSpec((blk, DIM), lambda i: (i, 0)),
        out_shape=jax.ShapeDtypeStruct((BATCH, DIM), jnp.float32),
    )(summed, idx, W1, b1, W2, b2)


def kernel(vectorized_text, table, W1, b1, W2, b2):
    idx = vectorized_text.astype(jnp.int32)
    table2 = _tc_convert(table)
    summed = _sc_embedding_sum(idx.reshape(-1), table2)
    return _tc_mlp(summed, idx, W1, b1.reshape(1, -1), W2, b2.reshape(1, -1))


# speedup vs baseline: 4.0335x; 1.0005x over previous
import functools

import jax
import jax.numpy as jnp
from jax import lax
from jax.experimental import pallas as pl
from jax.experimental.pallas import tpu as pltpu
from jax.experimental.pallas import tpu_sc as plsc

BATCH = 16384
HIST = 50
DIM = 64
VOCAB = 1000000

NUM_CORES = 2
NUM_SUBCORES = 16
NUM_WORKERS = NUM_CORES * NUM_SUBCORES
BAGS_PER_WORKER = BATCH // NUM_WORKERS
IDX_PER_WORKER = BAGS_PER_WORKER * HIST

CHUNK_BAGS = 8
CHUNK_ROWS = CHUNK_BAGS * HIST
SUB_SPLITS = ((0, 128), (128, 128), (256, 128), (384, 16))
NUM_CHUNKS = BAGS_PER_WORKER // CHUNK_BAGS

CONV_BLK = 24576


def _convert_kernel(tt_ref, out_ref):
    d = lax.broadcasted_iota(jnp.int32, (DIM, DIM), 0)
    o = lax.broadcasted_iota(jnp.int32, (DIM, DIM), 1)
    ident = (d == o).astype(jnp.float32)
    xt = lax.dot_general(tt_ref[...], ident, (((0,), (0,)), ((), ())),
                         preferred_element_type=jnp.float32)
    out_ref[...] = jnp.concatenate([xt, xt], axis=1)


def _tc_convert(table):
    tt = table.T
    grid = (pl.cdiv(VOCAB, CONV_BLK),)
    return pl.pallas_call(
        _convert_kernel,
        grid=grid,
        in_specs=[pl.BlockSpec((DIM, CONV_BLK), lambda i: (0, i))],
        out_specs=pl.BlockSpec((CONV_BLK, 2 * DIM), lambda i: (i, 0)),
        out_shape=jax.ShapeDtypeStruct((VOCAB, 2 * DIM), jnp.float32),
    )(tt)


def _sc_embedding_sum(idx_flat, table2):
    mesh = plsc.VectorSubcoreMesh(core_axis_name="c", subcore_axis_name="s")

    @functools.partial(
        pl.kernel,
        mesh=mesh,
        out_type=jax.ShapeDtypeStruct((BATCH, 2 * DIM), jnp.float32),
        scratch_types=[
            pltpu.VMEM((IDX_PER_WORKER,), jnp.int32),
            pltpu.VMEM((CHUNK_ROWS, 2 * DIM), jnp.float32),
            pltpu.VMEM((CHUNK_ROWS, 2 * DIM), jnp.float32),
            pltpu.VMEM((CHUNK_BAGS, 2 * DIM), jnp.float32),
            pltpu.SemaphoreType.DMA,
            pltpu.SemaphoreType.DMA,
        ],
    )
    def k(idx_hbm, table_hbm, out_hbm, idx_v, rows0, rows1, outb, sem0, sem1):
        wid = lax.axis_index("s") * NUM_CORES + lax.axis_index("c")
        bag0 = wid * BAGS_PER_WORKER

        pltpu.sync_copy(idx_hbm.at[pl.ds(bag0 * HIST, IDX_PER_WORKER)], idx_v)

        def start_gather(c, buf, sem):
            for off, n in SUB_SPLITS:
                pltpu.make_async_copy(
                    table_hbm.at[idx_v.at[pl.ds(c * CHUNK_ROWS + off, n)]],
                    buf.at[pl.ds(off, n)],
                    sem,
                ).start()

        def wait_gather(c, buf, sem):
            for off, n in SUB_SPLITS:
                pltpu.make_async_copy(
                    table_hbm.at[idx_v.at[pl.ds(c * CHUNK_ROWS + off, n)]],
                    buf.at[pl.ds(off, n)],
                    sem,
                ).wait()

        def accumulate(c, buf):
            for g in range(CHUNK_BAGS):
                def body(r, acc):
                    row = g * HIST + r
                    return tuple(
                        acc[j] + buf[row, pl.ds(j * 16, 16)] for j in range(4)
                    )
                zero = jnp.zeros((16,), jnp.float32)
                acc = lax.fori_loop(0, HIST, body, (zero, zero, zero, zero))
                for j in range(4):
                    outb[g, pl.ds(j * 16, 16)] = acc[j]
            pltpu.sync_copy(outb,
                            out_hbm.at[pl.ds(bag0 + c * CHUNK_BAGS, CHUNK_BAGS)])

        start_gather(0, rows0, sem0)
        start_gather(1, rows1, sem1)

        def loop_body(c2, carry):
            for b, (buf, sem) in enumerate(((rows0, sem0), (rows1, sem1))):
                c = c2 * 2 + b
                wait_gather(c, buf, sem)
                accumulate(c, buf)
                nxt = c + 2

                @pl.when(nxt < NUM_CHUNKS)
                def _():
                    start_gather(nxt, buf, sem)
            return carry

        lax.fori_loop(0, NUM_CHUNKS // 2, loop_body, 0)

    return k(idx_flat, table2)


def _tc_mlp_kernel(summed_ref, idx_ref, w1_ref, b1_ref, w2_ref, b2_ref, out_ref):
    counts = jnp.sum((idx_ref[...] != 0).astype(jnp.float32), axis=1,
                     keepdims=True)
    pooled = summed_ref[...][:, :DIM] / jnp.maximum(counts, 1.0)
    h = jnp.maximum(
        jnp.dot(pooled, w1_ref[...], preferred_element_type=jnp.float32)
        + b1_ref[...], 0.0)
    out_ref[...] = (
        jnp.dot(h, w2_ref[...], preferred_element_type=jnp.float32)
        + b2_ref[...])


def _tc_mlp(summed, idx, W1, b1, W2, b2):
    blk = 2048
    grid = (BATCH // blk,)
    return pl.pallas_call(
        _tc_mlp_kernel,
        grid=grid,
        in_specs=[
            pl.BlockSpec((blk, 2 * DIM), lambda i: (i, 0)),
            pl.BlockSpec((blk, HIST), lambda i: (i, 0)),
            pl.BlockSpec(W1.shape, lambda i: (0, 0)),
            pl.BlockSpec((1, b1.shape[1]), lambda i: (0, 0)),
            pl.BlockSpec(W2.shape, lambda i: (0, 0)),
            pl.BlockSpec((1, b2.shape[1]), lambda i: (0, 0)),
        ],
        out_specs=pl.BlockSpec((blk, DIM), lambda i: (i, 0)),
        out_shape=jax.ShapeDtypeStruct((BATCH, DIM), jnp.float32),
    )(summed, idx, W1, b1, W2, b2)


def kernel(vectorized_text, table, W1, b1, W2, b2):
    idx = vectorized_text.astype(jnp.int32)
    table2 = _tc_convert(table)
    summed = _sc_embedding_sum(idx.reshape(-1), table2)
    return _tc_mlp(summed, idx, W1, b1.reshape(1, -1), W2, b2.reshape(1, -1))


# speedup vs baseline: 5.2751x; 1.3078x over previous
import functools

import jax
import jax.numpy as jnp
from jax import lax
from jax.experimental import pallas as pl
from jax.experimental.pallas import tpu as pltpu
from jax.experimental.pallas import tpu_sc as plsc

BATCH = 16384
HIST = 50
DIM = 64
VOCAB = 1000000

NUM_CORES = 2
NUM_SUBCORES = 16
NUM_WORKERS = NUM_CORES * NUM_SUBCORES
BAGS_PER_WORKER = BATCH // NUM_WORKERS
IDX_PER_WORKER = BAGS_PER_WORKER * HIST

CHUNK_BAGS = 8
CHUNK_ROWS = CHUNK_BAGS * HIST
SUB_SPLITS = ((0, 128), (128, 128), (256, 128), (384, 16))
NUM_CHUNKS = BAGS_PER_WORKER // CHUNK_BAGS

CONV_BLK = 32768
CONV_HALF = CONV_BLK // 2
NUM_CONV_BLKS = pl.cdiv(VOCAB, CONV_BLK)
VOCAB_PAD = NUM_CONV_BLKS * CONV_BLK


def _convert_kernel(tt_ref, out_ref):
    d = lax.broadcasted_iota(jnp.int32, (DIM, DIM), 0)
    o = lax.broadcasted_iota(jnp.int32, (DIM, DIM), 1)
    ident = (d == o).astype(jnp.float32)
    xt = lax.dot_general(tt_ref[...], ident, (((0,), (0,)), ((), ())),
                         preferred_element_type=jnp.float32)
    y = jnp.concatenate([xt[:CONV_HALF], xt[CONV_HALF:]], axis=1)
    out_ref[...] = y.reshape(CONV_HALF * 2 * DIM)


def _tc_convert(table):
    tt = table.T
    return pl.pallas_call(
        _convert_kernel,
        grid=(NUM_CONV_BLKS,),
        in_specs=[pl.BlockSpec((DIM, CONV_BLK), lambda i: (0, i))],
        out_specs=pl.BlockSpec((CONV_BLK * DIM,), lambda i: (i,)),
        out_shape=jax.ShapeDtypeStruct((VOCAB_PAD * DIM,), jnp.float32),
    )(tt)


def _sc_embedding_sum(idx_flat, table_lin):
    mesh = plsc.VectorSubcoreMesh(core_axis_name="c", subcore_axis_name="s")

    @functools.partial(
        pl.kernel,
        mesh=mesh,
        compiler_params=pltpu.CompilerParams(use_tc_tiling_on_sc=False),
        out_type=jax.ShapeDtypeStruct((BATCH, DIM), jnp.float32),
        scratch_types=[
            pltpu.VMEM((IDX_PER_WORKER,), jnp.int32),
            pltpu.VMEM((CHUNK_ROWS, DIM), jnp.float32),
            pltpu.VMEM((CHUNK_ROWS, DIM), jnp.float32),
            pltpu.VMEM((CHUNK_BAGS, DIM), jnp.float32),
            pltpu.SemaphoreType.DMA,
            pltpu.SemaphoreType.DMA,
        ],
    )
    def k(idx_hbm, table_hbm, out_hbm, idx_v, rows0, rows1, outb, sem0, sem1):
        wid = lax.axis_index("s") * NUM_CORES + lax.axis_index("c")
        bag0 = wid * BAGS_PER_WORKER

        pltpu.sync_copy(idx_hbm.at[pl.ds(bag0 * HIST, IDX_PER_WORKER)], idx_v)

        def remap_body(i, carry):
            v = idx_v[pl.ds(i * 16, 16)]
            f = ((v & jnp.int32(-CONV_BLK))
                 | ((v & jnp.int32(CONV_HALF - 1)) << 1)
                 | ((v >> jnp.int32(14)) & jnp.int32(1)))
            idx_v[pl.ds(i * 16, 16)] = f
            return carry

        lax.fori_loop(0, IDX_PER_WORKER // 16, remap_body, 0)

        def start_gather(c, buf, sem):
            for off, n in SUB_SPLITS:
                pltpu.make_async_copy(
                    table_hbm.at[idx_v.at[pl.ds(c * CHUNK_ROWS + off, n)]],
                    buf.at[pl.ds(off, n)],
                    sem,
                ).start()

        def wait_gather(c, buf, sem):
            for off, n in SUB_SPLITS:
                pltpu.make_async_copy(
                    table_hbm.at[idx_v.at[pl.ds(c * CHUNK_ROWS + off, n)]],
                    buf.at[pl.ds(off, n)],
                    sem,
                ).wait()

        def accumulate(c, buf):
            for g in range(CHUNK_BAGS):
                def body(r, acc):
                    row = g * HIST + r
                    return tuple(
                        acc[j] + buf[row, pl.ds(j * 16, 16)] for j in range(4)
                    )
                zero = jnp.zeros((16,), jnp.float32)
                acc = lax.fori_loop(0, HIST, body, (zero, zero, zero, zero))
                for j in range(4):
                    outb[g, pl.ds(j * 16, 16)] = acc[j]
            pltpu.sync_copy(outb,
                            out_hbm.at[pl.ds(bag0 + c * CHUNK_BAGS, CHUNK_BAGS)])

        start_gather(0, rows0, sem0)
        start_gather(1, rows1, sem1)

        def loop_body(c2, carry):
            for b, (buf, sem) in enumerate(((rows0, sem0), (rows1, sem1))):
                c = c2 * 2 + b
                wait_gather(c, buf, sem)
                accumulate(c, buf)
                nxt = c + 2

                @pl.when(nxt < NUM_CHUNKS)
                def _():
                    start_gather(nxt, buf, sem)
            return carry

        lax.fori_loop(0, NUM_CHUNKS // 2, loop_body, 0)

    return k(idx_flat, table_lin)


def _tc_mlp_kernel(summed_ref, idx_ref, w1_ref, b1_ref, w2_ref, b2_ref, out_ref):
    counts = jnp.sum((idx_ref[...] != 0).astype(jnp.float32), axis=1,
                     keepdims=True)
    pooled = summed_ref[...] / jnp.maximum(counts, 1.0)
    h = jnp.maximum(
        jnp.dot(pooled, w1_ref[...], preferred_element_type=jnp.float32)
        + b1_ref[...], 0.0)
    out_ref[...] = (
        jnp.dot(h, w2_ref[...], preferred_element_type=jnp.float32)
        + b2_ref[...])


def _tc_mlp(summed, idx, W1, b1, W2, b2):
    blk = 2048
    grid = (BATCH // blk,)
    return pl.pallas_call(
        _tc_mlp_kernel,
        grid=grid,
        in_specs=[
            pl.BlockSpec((blk, DIM), lambda i: (i, 0)),
            pl.BlockSpec((blk, HIST), lambda i: (i, 0)),
            pl.BlockSpec(W1.shape, lambda i: (0, 0)),
            pl.BlockSpec((1, b1.shape[1]), lambda i: (0, 0)),
            pl.BlockSpec(W2.shape, lambda i: (0, 0)),
            pl.BlockSpec((1, b2.shape[1]), lambda i: (0, 0)),
        ],
        out_specs=pl.BlockSpec((blk, DIM), lambda i: (i, 0)),
        out_shape=jax.ShapeDtypeStruct((BATCH, DIM), jnp.float32),
    )(summed, idx, W1, b1, W2, b2)


def kernel(vectorized_text, table, W1, b1, W2, b2):
    idx = vectorized_text.astype(jnp.int32)
    table_lin = _tc_convert(table).reshape(VOCAB_PAD, DIM)
    summed = _sc_embedding_sum(idx.reshape(-1), table_lin)
    return _tc_mlp(summed, idx, W1, b1.reshape(1, -1), W2, b2.reshape(1, -1))


# speedup vs baseline: 5.5201x; 1.0464x over previous
import functools

import jax
import jax.numpy as jnp
from jax import lax
from jax.experimental import pallas as pl
from jax.experimental.pallas import tpu as pltpu
from jax.experimental.pallas import tpu_sc as plsc

BATCH = 16384
HIST = 50
DIM = 64
VOCAB = 1000000

NUM_CORES = 2
NUM_SUBCORES = 16
NUM_WORKERS = NUM_CORES * NUM_SUBCORES
BAGS_PER_WORKER = BATCH // NUM_WORKERS
IDX_PER_WORKER = BAGS_PER_WORKER * HIST

CHUNK_BAGS = 8
CHUNK_ROWS = CHUNK_BAGS * HIST
SUB_SPLITS = ((0, 128), (128, 128), (256, 128), (384, 16))
NUM_CHUNKS = BAGS_PER_WORKER // CHUNK_BAGS

CONV_BLK = 32768
CONV_HALF = CONV_BLK // 2
NUM_CONV_BLKS = pl.cdiv(VOCAB, CONV_BLK)
VOCAB_PAD = NUM_CONV_BLKS * CONV_BLK


def _convert_kernel(tt_ref, out_ref):
    d = lax.broadcasted_iota(jnp.int32, (DIM, DIM), 0)
    o = lax.broadcasted_iota(jnp.int32, (DIM, DIM), 1)
    ident = (d == o).astype(jnp.float32)
    xt = lax.dot_general(tt_ref[...], ident, (((0,), (0,)), ((), ())),
                         preferred_element_type=jnp.float32)
    y = jnp.concatenate([xt[:CONV_HALF], xt[CONV_HALF:]], axis=1)
    out_ref[...] = y.reshape(CONV_HALF * 2 * DIM)


def _tc_convert(table):
    tt = table.T
    return pl.pallas_call(
        _convert_kernel,
        grid=(NUM_CONV_BLKS,),
        in_specs=[pl.BlockSpec((DIM, CONV_BLK), lambda i: (0, i))],
        out_specs=pl.BlockSpec((CONV_BLK * DIM,), lambda i: (i,)),
        out_shape=jax.ShapeDtypeStruct((VOCAB_PAD * DIM,), jnp.float32),
    )(tt)


def _sc_embedding_sum(idx_flat, table_lin):
    mesh = plsc.VectorSubcoreMesh(core_axis_name="c", subcore_axis_name="s")

    @functools.partial(
        pl.kernel,
        mesh=mesh,
        compiler_params=pltpu.CompilerParams(use_tc_tiling_on_sc=False),
        out_type=jax.ShapeDtypeStruct((BATCH, DIM), jnp.float32),
        scratch_types=[
            pltpu.VMEM((IDX_PER_WORKER,), jnp.int32),
            pltpu.VMEM((CHUNK_ROWS, DIM), jnp.float32),
            pltpu.VMEM((CHUNK_ROWS, DIM), jnp.float32),
            pltpu.VMEM((CHUNK_BAGS, DIM), jnp.float32),
            pltpu.SemaphoreType.DMA,
            pltpu.SemaphoreType.DMA,
        ],
    )
    def k(idx_hbm, table_hbm, out_hbm, idx_v, rows0, rows1, outb, sem0, sem1):
        wid = lax.axis_index("s") * NUM_CORES + lax.axis_index("c")
        bag0 = wid * BAGS_PER_WORKER

        pltpu.sync_copy(idx_hbm.at[pl.ds(bag0 * HIST, IDX_PER_WORKER)], idx_v)

        def remap_body(i, carry):
            v = idx_v[pl.ds(i * 16, 16)]
            f = ((v & jnp.int32(-CONV_BLK))
                 | ((v & jnp.int32(CONV_HALF - 1)) << 1)
                 | ((v >> jnp.int32(14)) & jnp.int32(1)))
            idx_v[pl.ds(i * 16, 16)] = f
            return carry

        lax.fori_loop(0, IDX_PER_WORKER // 16, remap_body, 0)

        def start_gather(c, buf, sem):
            for off, n in SUB_SPLITS:
                pltpu.make_async_copy(
                    table_hbm.at[idx_v.at[pl.ds(c * CHUNK_ROWS + off, n)]],
                    buf.at[pl.ds(off, n)],
                    sem,
                ).start()

        def wait_gather(c, buf, sem):
            for off, n in SUB_SPLITS:
                pltpu.make_async_copy(
                    table_hbm.at[idx_v.at[pl.ds(c * CHUNK_ROWS + off, n)]],
                    buf.at[pl.ds(off, n)],
                    sem,
                ).wait()

        def accumulate(c, buf):
            for g in range(CHUNK_BAGS):
                def body(r2, acc):
                    row = g * HIST + r2 * 2
                    return tuple(
                        acc[j]
                        + buf[row, pl.ds(j * 16, 16)]
                        + buf[row + 1, pl.ds(j * 16, 16)]
                        for j in range(4)
                    )
                zero = jnp.zeros((16,), jnp.float32)
                acc = lax.fori_loop(0, HIST // 2, body, (zero, zero, zero, zero))
                for j in range(4):
                    outb[g, pl.ds(j * 16, 16)] = acc[j]
            pltpu.sync_copy(outb,
                            out_hbm.at[pl.ds(bag0 + c * CHUNK_BAGS, CHUNK_BAGS)])

        start_gather(0, rows0, sem0)
        start_gather(1, rows1, sem1)

        def loop_body(c2, carry):
            for b, (buf, sem) in enumerate(((rows0, sem0), (rows1, sem1))):
                c = c2 * 2 + b
                wait_gather(c, buf, sem)
                accumulate(c, buf)
                nxt = c + 2

                @pl.when(nxt < NUM_CHUNKS)
                def _():
                    start_gather(nxt, buf, sem)
            return carry

        lax.fori_loop(0, NUM_CHUNKS // 2, loop_body, 0)

    return k(idx_flat, table_lin)


def _tc_mlp_kernel(summed_ref, idx_ref, w1_ref, b1_ref, w2_ref, b2_ref, out_ref):
    counts = jnp.sum((idx_ref[...] != 0).astype(jnp.float32), axis=1,
                     keepdims=True)
    pooled = summed_ref[...] / jnp.maximum(counts, 1.0)
    h = jnp.maximum(
        jnp.dot(pooled, w1_ref[...], preferred_element_type=jnp.float32)
        + b1_ref[...], 0.0)
    out_ref[...] = (
        jnp.dot(h, w2_ref[...], preferred_element_type=jnp.float32)
        + b2_ref[...])


def _tc_mlp(summed, idx, W1, b1, W2, b2):
    blk = 2048
    grid = (BATCH // blk,)
    return pl.pallas_call(
        _tc_mlp_kernel,
        grid=grid,
        in_specs=[
            pl.BlockSpec((blk, DIM), lambda i: (i, 0)),
            pl.BlockSpec((blk, HIST), lambda i: (i, 0)),
            pl.BlockSpec(W1.shape, lambda i: (0, 0)),
            pl.BlockSpec((1, b1.shape[1]), lambda i: (0, 0)),
            pl.BlockSpec(W2.shape, lambda i: (0, 0)),
            pl.BlockSpec((1, b2.shape[1]), lambda i: (0, 0)),
        ],
        out_specs=pl.BlockSpec((blk, DIM), lambda i: (i, 0)),
        out_shape=jax.ShapeDtypeStruct((BATCH, DIM), jnp.float32),
    )(summed, idx, W1, b1, W2, b2)


def kernel(vectorized_text, table, W1, b1, W2, b2):
    idx = vectorized_text.astype(jnp.int32)
    table_lin = _tc_convert(table).reshape(VOCAB_PAD, DIM)
    summed = _sc_embedding_sum(idx.reshape(-1), table_lin)
    return _tc_mlp(summed, idx, W1, b1.reshape(1, -1), W2, b2.reshape(1, -1))
